# Initial kernel scaffold; baseline (speedup 1.0000x reference)
import jax
import jax.numpy as jnp
from jax.experimental import pallas as pl


def kernel(x, edge_index, edge_weight, block_mask, batch, params):
    raise NotImplementedError("write your pallas kernel here")


# speedup vs baseline: 1.1246x; 1.1246x over previous
import functools

import jax
import jax.numpy as jnp
from jax.experimental import pallas as pl

_N = 50000
_H = 64
_G = 128
_RATIO = 0.8
_BLK = 1000
_NB = _N // _BLK
_EPS = 1e-5


def _row(width=_H):
    return pl.BlockSpec((_BLK, width), lambda i: (i, 0))


def _full(shape):
    return pl.BlockSpec(shape, lambda i: (0,) * len(shape))


def _elu(v):
    return jnp.where(v > 0, v, jnp.exp(jnp.minimum(v, 0.0)) - 1.0)


def _dot(a, b):
    return jnp.dot(a, b, preferred_element_type=jnp.float32)


def _k1_body(h0, w0, b0, wg, xw_out):
    h1 = jnp.maximum(_dot(h0[...], w0[...]) + b0[...], 0.0)
    xw_out[...] = _dot(h1, wg[...])


def _mom_body(x, out):
    @pl.when(pl.program_id(0) == 0)
    def _():
        out[...] = jnp.zeros_like(out)

    xb = x[...]
    out[0:1, :] += jnp.sum(xb, axis=0, keepdims=True)
    out[1:2, :] += jnp.sum(xb * xb, axis=0, keepdims=True)


def _k2a_body(te, xw, d2, bg, out):
    out[...] = te[...] + d2[...] * xw[...] + bg[...]


def _k2b_body(tf, h0, a0, c0, w1a, w1b, b1, out):
    h2 = a0[...] * tf[...] + c0[...]
    out[...] = _dot(h2, w1a[...]) + _dot(h0[...], w1b[...]) + b1[...]


def _ga1_body(h, bmf, wt0, bt0, wt1, bt1, out):
    hh = h[...]
    x0 = _elu(_dot(hh, wt0[...]) + bt0[...])
    x1 = _elu(_dot(hh, wt1[...]) + bt1[...])
    out[...] = (_RATIO * x0 + (1.0 - _RATIO) * x1
                + bmf[...] * ((2.0 * _RATIO - 1.0) * (x1 - x0)))


def _ga2_body(h, bmf, a1, c1, wt0, bt0, wt1, bt1, xm_out, h5_out):
    h5 = _elu(a1[...] * h[...] + c1[...])
    h5_out[...] = h5
    x0 = _elu(_dot(h5, wt0[...]) + bt0[...])
    x1 = _elu(_dot(h5, wt1[...]) + bt1[...])
    xm_out[...] = (_RATIO * x0 + (1.0 - _RATIO) * x1
                   + bmf[...] * ((2.0 * _RATIO - 1.0) * (x1 - x0)))


def _gb_body(ag, origin, bmf, a, c, wc0a, wc0b, bc0, wc1a, wc1b, bc1, out):
    xn = a[...] * ag[...] + c[...]
    org = origin[...]
    y0 = _dot(xn, wc0a[...]) + _dot(org, wc0b[...]) + bc0[...]
    y1 = _dot(xn, wc1a[...]) + _dot(org, wc1b[...]) + bc1[...]
    out[...] = (_RATIO * y0 + (1.0 - _RATIO) * y1
                + bmf[...] * ((2.0 * _RATIO - 1.0) * (y1 - y0)))


def _k3_body(h4, h6, bmf, a2a, c2a, a2b, c2b, out):
    m = bmf[...]
    out[...] = jnp.concatenate(
        [m * (a2a[...] * h4[...] + c2a[...]),
         m * (a2b[...] * h6[...] + c2b[...])], axis=1)


def _k4_body(pooled, wout, bout, out):
    out[...] = _dot(pooled[...], wout[...]) + bout[...]


def _moments(x, width=_H):
    out = pl.pallas_call(
        _mom_body,
        grid=(_NB,),
        in_specs=[_row(width)],
        out_specs=_full((8, width)),
        out_shape=jax.ShapeDtypeStruct((8, width), jnp.float32),
    )(x)
    return out[0], out[1]


def _gn_affine(x, w, b, ms, width=_H):
    s1, s2 = _moments(x, width)
    mean = s1 / _N
    m2 = s2 / _N
    var = m2 - mean * mean * ms * (2.0 - ms)
    a = w / jnp.sqrt(var + _EPS)
    c = b - a * mean * ms
    return a.reshape(1, width), c.reshape(1, width)


def _r2(v):
    return v.reshape(1, -1)


@jax.jit
def kernel(x, edge_index, edge_weight, block_mask, batch, params):
    p = params
    row_i = edge_index[0]
    col_i = edge_index[1]
    bmf = (block_mask > 0).astype(jnp.float32).reshape(_N, 1)

    ones_e = jnp.ones(row_i.shape[0], jnp.float32)
    deg = jax.ops.segment_sum(ones_e, col_i, _N) + 1.0
    dinv = 1.0 / jnp.sqrt(deg)
    norm_e = dinv[row_i] * dinv[col_i]
    d2 = (dinv * dinv).reshape(_N, 1)

    col_sum = jax.ops.segment_sum(edge_weight, col_i, _N)
    col_sum = jnp.where(col_sum < 1, col_sum + 1, col_sum)
    aggr_w = (1.0 / col_sum)[row_i] * edge_weight

    h0 = p["emb"][x[:, 0]]

    xw = pl.pallas_call(
        _k1_body,
        grid=(_NB,),
        in_specs=[_row(), _full((_H, _H)), _full((1, _H)), _full((_H, _H))],
        out_specs=_row(),
        out_shape=jax.ShapeDtypeStruct((_N, _H), jnp.float32),
    )(h0, p["W0"], _r2(p["b0"]), p["Wg"])

    te = jax.ops.segment_sum(norm_e[:, None] * xw[row_i], col_i, _N)
    tf = pl.pallas_call(
        _k2a_body,
        grid=(_NB,),
        in_specs=[_row(), _row(), _row(1), _full((1, _H))],
        out_specs=_row(),
        out_shape=jax.ShapeDtypeStruct((_N, _H), jnp.float32),
    )(te, xw, d2, _r2(p["bg"]))

    a0, c0 = _gn_affine(tf, p["gn0_w"], p["gn0_b"], p["gn0_ms"])
    h3 = pl.pallas_call(
        _k2b_body,
        grid=(_NB,),
        in_specs=[_row(), _row(), _full((1, _H)), _full((1, _H)),
                  _full((_H, _H)), _full((_H, _H)), _full((1, _H))],
        out_specs=_row(),
        out_shape=jax.ShapeDtypeStruct((_N, _H), jnp.float32),
    )(tf, h0, a0, c0, p["W1"][:_H], p["W1"][_H:], _r2(p["b1"]))

    g1 = p["g1"]
    xm = pl.pallas_call(
        _ga1_body,
        grid=(_NB,),
        in_specs=[_row(), _row(1), _full((_H, _H)), _full((1, _H)),
                  _full((_H, _H)), _full((1, _H))],
        out_specs=_row(),
        out_shape=jax.ShapeDtypeStruct((_N, _H), jnp.float32),
    )(h3, bmf, g1["Wt0"], _r2(g1["bt0"]), g1["Wt1"], _r2(g1["bt1"]))

    ag = jax.ops.segment_sum(aggr_w[:, None] * xm[col_i], row_i, _N)
    a_g1, c_g1 = _gn_affine(ag, g1["gn_w"], g1["gn_b"], g1["gn_ms"])

    def _gb_call(agx, origin, a, c, gp):
        return pl.pallas_call(
            _gb_body,
            grid=(_NB,),
            in_specs=[_row(), _row(), _row(1), _full((1, _H)), _full((1, _H)),
                      _full((_H, _H)), _full((_H, _H)), _full((1, _H)),
                      _full((_H, _H)), _full((_H, _H)), _full((1, _H))],
            out_specs=_row(),
            out_shape=jax.ShapeDtypeStruct((_N, _H), jnp.float32),
        )(agx, origin, bmf, a, c,
          gp["Wc0"][:_H], gp["Wc0"][_H:], _r2(gp["bc0"]),
          gp["Wc1"][:_H], gp["Wc1"][_H:], _r2(gp["bc1"]))

    h4 = _gb_call(ag, h3, a_g1, c_g1, g1)

    a1, c1 = _gn_affine(h4, p["gn1_w"], p["gn1_b"], p["gn1_ms"])
    g2 = p["g2"]
    xm2, h5 = pl.pallas_call(
        _ga2_body,
        grid=(_NB,),
        in_specs=[_row(), _row(1), _full((1, _H)), _full((1, _H)),
                  _full((_H, _H)), _full((1, _H)), _full((_H, _H)),
                  _full((1, _H))],
        out_specs=[_row(), _row()],
        out_shape=[jax.ShapeDtypeStruct((_N, _H), jnp.float32),
                   jax.ShapeDtypeStruct((_N, _H), jnp.float32)],
    )(h4, bmf, a1, c1, g2["Wt0"], _r2(g2["bt0"]), g2["Wt1"], _r2(g2["bt1"]))

    ag2 = jax.ops.segment_sum(aggr_w[:, None] * xm2[col_i], row_i, _N)
    a_g2, c_g2 = _gn_affine(ag2, g2["gn_w"], g2["gn_b"], g2["gn_ms"])
    h6 = _gb_call(ag2, h5, a_g2, c_g2, g2)

    a2a, c2a = _gn_affine(h4, p["gn2_w"][:_H], p["gn2_b"][:_H], p["gn2_ms"][:_H])
    a2b, c2b = _gn_affine(h6, p["gn2_w"][_H:], p["gn2_b"][_H:], p["gn2_ms"][_H:])
    h7 = pl.pallas_call(
        _k3_body,
        grid=(_NB,),
        in_specs=[_row(), _row(), _row(1), _full((1, _H)), _full((1, _H)),
                  _full((1, _H)), _full((1, _H))],
        out_specs=_row(2 * _H),
        out_shape=jax.ShapeDtypeStruct((_N, 2 * _H), jnp.float32),
    )(h4, h6, bmf, a2a, c2a, a2b, c2b)

    sums = jax.ops.segment_sum(h7, batch, _G)
    counts = jax.ops.segment_sum(jnp.ones(_N, jnp.float32), batch, _G)
    pooled = sums / jnp.maximum(counts, 1.0)[:, None]

    out = pl.pallas_call(
        _k4_body,
        in_specs=[pl.BlockSpec((_G, 2 * _H), lambda: (0, 0)),
                  pl.BlockSpec((2 * _H, _H), lambda: (0, 0)),
                  pl.BlockSpec((1, _H), lambda: (0, 0))],
        out_specs=pl.BlockSpec((_G, _H), lambda: (0, 0)),
        out_shape=jax.ShapeDtypeStruct((_G, _H), jnp.float32),
    )(pooled, p["Wout"], _r2(p["bout"]))
    return out


# speedup vs baseline: 1.1988x; 1.0659x over previous
import functools

import jax
import jax.numpy as jnp
from jax import lax
from jax.experimental import pallas as pl
from jax.experimental.pallas import tpu as pltpu
from jax.experimental.pallas import tpu_sc as plsc

_N = 50000
_H = 64
_G = 128
_RATIO = 0.8
_BLK = 1000
_NB = _N // _BLK
_EPS = 1e-5


def _row(width=_H):
    return pl.BlockSpec((_BLK, width), lambda i: (i, 0))


def _full(shape):
    return pl.BlockSpec(shape, lambda i: (0,) * len(shape))


def _elu(v):
    return jnp.where(v > 0, v, jnp.exp(jnp.minimum(v, 0.0)) - 1.0)


def _dot(a, b):
    return jnp.dot(a, b, preferred_element_type=jnp.float32)


def _k1_body(h0, w0, b0, wg, xw_out):
    h1 = jnp.maximum(_dot(h0[...], w0[...]) + b0[...], 0.0)
    xw_out[...] = _dot(h1, wg[...])


def _mom_body(x, out):
    @pl.when(pl.program_id(0) == 0)
    def _():
        out[...] = jnp.zeros_like(out)

    xb = x[...]
    out[0:1, :] += jnp.sum(xb, axis=0, keepdims=True)
    out[1:2, :] += jnp.sum(xb * xb, axis=0, keepdims=True)


def _k2a_body(te, xw, d2, bg, out):
    out[...] = te[...] + d2[...] * xw[...] + bg[...]


def _k2b_body(tf, h0, a0, c0, w1a, w1b, b1, out):
    h2 = a0[...] * tf[...] + c0[...]
    out[...] = _dot(h2, w1a[...]) + _dot(h0[...], w1b[...]) + b1[...]


def _ga1_body(h, bmf, wt0, bt0, wt1, bt1, out):
    hh = h[...]
    x0 = _elu(_dot(hh, wt0[...]) + bt0[...])
    x1 = _elu(_dot(hh, wt1[...]) + bt1[...])
    out[...] = (_RATIO * x0 + (1.0 - _RATIO) * x1
                + bmf[...] * ((2.0 * _RATIO - 1.0) * (x1 - x0)))


def _ga2_body(h, bmf, a1, c1, wt0, bt0, wt1, bt1, xm_out, h5_out):
    h5 = _elu(a1[...] * h[...] + c1[...])
    h5_out[...] = h5
    x0 = _elu(_dot(h5, wt0[...]) + bt0[...])
    x1 = _elu(_dot(h5, wt1[...]) + bt1[...])
    xm_out[...] = (_RATIO * x0 + (1.0 - _RATIO) * x1
                   + bmf[...] * ((2.0 * _RATIO - 1.0) * (x1 - x0)))


def _gb_body(ag, origin, bmf, a, c, wc0a, wc0b, bc0, wc1a, wc1b, bc1, out):
    xn = a[...] * ag[...] + c[...]
    org = origin[...]
    y0 = _dot(xn, wc0a[...]) + _dot(org, wc0b[...]) + bc0[...]
    y1 = _dot(xn, wc1a[...]) + _dot(org, wc1b[...]) + bc1[...]
    out[...] = (_RATIO * y0 + (1.0 - _RATIO) * y1
                + bmf[...] * ((2.0 * _RATIO - 1.0) * (y1 - y0)))


def _k3_body(h4, h6, bmf, a2a, c2a, a2b, c2b, out):
    m = bmf[...]
    out[...] = jnp.concatenate(
        [m * (a2a[...] * h4[...] + c2a[...]),
         m * (a2b[...] * h6[...] + c2b[...])], axis=1)


def _k4_body(pooled, wout, bout, out):
    out[...] = _dot(pooled[...], wout[...]) + bout[...]


def _moments(x, width=_H):
    out = pl.pallas_call(
        _mom_body,
        grid=(_NB,),
        in_specs=[_row(width)],
        out_specs=_full((8, width)),
        out_shape=jax.ShapeDtypeStruct((8, width), jnp.float32),
    )(x)
    return out[0], out[1]


def _gn_affine(x, w, b, ms, width=_H):
    s1, s2 = _moments(x, width)
    mean = s1 / _N
    m2 = s2 / _N
    var = m2 - mean * mean * ms * (2.0 - ms)
    a = w / jnp.sqrt(var + _EPS)
    c = b - a * mean * ms
    return a.reshape(1, width), c.reshape(1, width)


def _r2(v):
    return v.reshape(1, -1)


_NW = 32
_CH = 1000


def _sc_gather(table, idx):
    e_tot = idx.shape[0]
    b_per_w = e_tot // _NW
    n_ch = b_per_w // _CH
    mesh = plsc.VectorSubcoreMesh(core_axis_name="c", subcore_axis_name="s")
    table = jnp.pad(table, ((0, 0), (0, 2 * _H - _H)))

    @functools.partial(
        pl.kernel,
        mesh=mesh,
        out_type=jax.ShapeDtypeStruct((e_tot, 2 * _H), jnp.float32),
        scratch_types=[
            pltpu.VMEM((_CH,), jnp.int32),
            pltpu.VMEM((_CH, 2 * _H), jnp.float32),
            pltpu.SemaphoreType.DMA,
        ],
    )
    def k(table_hbm, idx_hbm, out_hbm, idx_v, rows_v, sem):
        wid = lax.axis_index("s") * 2 + lax.axis_index("c")
        base = wid * b_per_w

        def body(j, carry):
            off = base + j * _CH
            pltpu.sync_copy(idx_hbm.at[pl.ds(off, _CH)], idx_v)
            pltpu.async_copy(table_hbm.at[idx_v], rows_v, sem).wait()
            pltpu.sync_copy(rows_v, out_hbm.at[pl.ds(off, _CH)])
            return carry

        lax.fori_loop(0, n_ch, body, 0)

    return k(table, idx)[:, :_H]


@jax.jit
def kernel(x, edge_index, edge_weight, block_mask, batch, params):
    p = params
    row_i = edge_index[0]
    col_i = edge_index[1]
    bmf = (block_mask > 0).astype(jnp.float32).reshape(_N, 1)

    ones_e = jnp.ones(row_i.shape[0], jnp.float32)
    deg = jax.ops.segment_sum(ones_e, col_i, _N) + 1.0
    dinv = 1.0 / jnp.sqrt(deg)
    norm_e = dinv[row_i] * dinv[col_i]
    d2 = (dinv * dinv).reshape(_N, 1)

    col_sum = jax.ops.segment_sum(edge_weight, col_i, _N)
    col_sum = jnp.where(col_sum < 1, col_sum + 1, col_sum)
    aggr_w = (1.0 / col_sum)[row_i] * edge_weight

    h0 = p["emb"][x[:, 0]]

    xw = pl.pallas_call(
        _k1_body,
        grid=(_NB,),
        in_specs=[_row(), _full((_H, _H)), _full((1, _H)), _full((_H, _H))],
        out_specs=_row(),
        out_shape=jax.ShapeDtypeStruct((_N, _H), jnp.float32),
    )(h0, p["W0"], _r2(p["b0"]), p["Wg"])

    te = jax.ops.segment_sum(norm_e[:, None] * xw[row_i], col_i, _N)
    tf = pl.pallas_call(
        _k2a_body,
        grid=(_NB,),
        in_specs=[_row(), _row(), _row(1), _full((1, _H))],
        out_specs=_row(),
        out_shape=jax.ShapeDtypeStruct((_N, _H), jnp.float32),
    )(te, xw, d2, _r2(p["bg"]))

    a0, c0 = _gn_affine(tf, p["gn0_w"], p["gn0_b"], p["gn0_ms"])
    h3 = pl.pallas_call(
        _k2b_body,
        grid=(_NB,),
        in_specs=[_row(), _row(), _full((1, _H)), _full((1, _H)),
                  _full((_H, _H)), _full((_H, _H)), _full((1, _H))],
        out_specs=_row(),
        out_shape=jax.ShapeDtypeStruct((_N, _H), jnp.float32),
    )(tf, h0, a0, c0, p["W1"][:_H], p["W1"][_H:], _r2(p["b1"]))

    g1 = p["g1"]
    xm = pl.pallas_call(
        _ga1_body,
        grid=(_NB,),
        in_specs=[_row(), _row(1), _full((_H, _H)), _full((1, _H)),
                  _full((_H, _H)), _full((1, _H))],
        out_specs=_row(),
        out_shape=jax.ShapeDtypeStruct((_N, _H), jnp.float32),
    )(h3, bmf, g1["Wt0"], _r2(g1["bt0"]), g1["Wt1"], _r2(g1["bt1"]))

    ag = jax.ops.segment_sum(aggr_w[:, None] * _sc_gather(xm, col_i), row_i, _N)
    a_g1, c_g1 = _gn_affine(ag, g1["gn_w"], g1["gn_b"], g1["gn_ms"])

    def _gb_call(agx, origin, a, c, gp):
        return pl.pallas_call(
            _gb_body,
            grid=(_NB,),
            in_specs=[_row(), _row(), _row(1), _full((1, _H)), _full((1, _H)),
                      _full((_H, _H)), _full((_H, _H)), _full((1, _H)),
                      _full((_H, _H)), _full((_H, _H)), _full((1, _H))],
            out_specs=_row(),
            out_shape=jax.ShapeDtypeStruct((_N, _H), jnp.float32),
        )(agx, origin, bmf, a, c,
          gp["Wc0"][:_H], gp["Wc0"][_H:], _r2(gp["bc0"]),
          gp["Wc1"][:_H], gp["Wc1"][_H:], _r2(gp["bc1"]))

    h4 = _gb_call(ag, h3, a_g1, c_g1, g1)

    a1, c1 = _gn_affine(h4, p["gn1_w"], p["gn1_b"], p["gn1_ms"])
    g2 = p["g2"]
    xm2, h5 = pl.pallas_call(
        _ga2_body,
        grid=(_NB,),
        in_specs=[_row(), _row(1), _full((1, _H)), _full((1, _H)),
                  _full((_H, _H)), _full((1, _H)), _full((_H, _H)),
                  _full((1, _H))],
        out_specs=[_row(), _row()],
        out_shape=[jax.ShapeDtypeStruct((_N, _H), jnp.float32),
                   jax.ShapeDtypeStruct((_N, _H), jnp.float32)],
    )(h4, bmf, a1, c1, g2["Wt0"], _r2(g2["bt0"]), g2["Wt1"], _r2(g2["bt1"]))

    ag2 = jax.ops.segment_sum(aggr_w[:, None] * _sc_gather(xm2, col_i), row_i, _N)
    a_g2, c_g2 = _gn_affine(ag2, g2["gn_w"], g2["gn_b"], g2["gn_ms"])
    h6 = _gb_call(ag2, h5, a_g2, c_g2, g2)

    a2a, c2a = _gn_affine(h4, p["gn2_w"][:_H], p["gn2_b"][:_H], p["gn2_ms"][:_H])
    a2b, c2b = _gn_affine(h6, p["gn2_w"][_H:], p["gn2_b"][_H:], p["gn2_ms"][_H:])
    h7 = pl.pallas_call(
        _k3_body,
        grid=(_NB,),
        in_specs=[_row(), _row(), _row(1), _full((1, _H)), _full((1, _H)),
                  _full((1, _H)), _full((1, _H))],
        out_specs=_row(2 * _H),
        out_shape=jax.ShapeDtypeStruct((_N, 2 * _H), jnp.float32),
    )(h4, h6, bmf, a2a, c2a, a2b, c2b)

    sums = jax.ops.segment_sum(h7, batch, _G)
    counts = jax.ops.segment_sum(jnp.ones(_N, jnp.float32), batch, _G)
    pooled = sums / jnp.maximum(counts, 1.0)[:, None]

    out = pl.pallas_call(
        _k4_body,
        in_specs=[pl.BlockSpec((_G, 2 * _H), lambda: (0, 0)),
                  pl.BlockSpec((2 * _H, _H), lambda: (0, 0)),
                  pl.BlockSpec((1, _H), lambda: (0, 0))],
        out_specs=pl.BlockSpec((_G, _H), lambda: (0, 0)),
        out_shape=jax.ShapeDtypeStruct((_G, _H), jnp.float32),
    )(pooled, p["Wout"], _r2(p["bout"]))
    return out
